# baseline (speedup 1.0000x reference)
import functools

import jax
import jax.numpy as jnp
from jax import lax
from jax.experimental import pallas as pl
from jax.experimental.pallas import tpu as pltpu
from jax.experimental.pallas import tpu_sc as plsc

SEQ = 4096
BATCH = 4
D_MODEL = 1024
N_ROWS = SEQ * BATCH
_L = 16
_NC, _NS = 2, 16
_NW = _NC * _NS
_B_PER_W = N_ROWS // _NW
_C = 16
_CHUNKS = _B_PER_W // _C
_NPAIR = _CHUNKS // 2


def _sc_body(pe_hbm, idx_hbm, x_hbm, out_hbm, idx_v,
             pe_v0, pe_v1, o_v0, o_v1,
             sx0, sx1, sg0, sg1, so0, so1):
    wid = lax.axis_index("s") * _NC + lax.axis_index("c")
    xf = x_hbm.reshape(N_ROWS, D_MODEL)
    of = out_hbm.reshape(N_ROWS, D_MODEL)
    rbase_w = wid * _B_PER_W
    pltpu.sync_copy(idx_hbm.at[wid], idx_v)

    pes = (pe_v0, pe_v1)
    outs = (o_v0, o_v1)
    sxs = (sx0, sx1)
    sgs = (sg0, sg1)
    sos = (so0, so1)

    def issue_loads(c, s):
        pltpu.async_copy(xf.at[pl.ds(rbase_w + c * _C, _C)], outs[s], sxs[s])
        pltpu.async_copy(pe_hbm.at[idx_v.at[c]], pes[s], sgs[s])

    def wait_loads(s):
        pltpu.make_async_copy(xf.at[pl.ds(0, _C)], outs[s], sxs[s]).wait()
        pltpu.make_async_copy(pe_hbm.at[idx_v.at[0]], pes[s], sgs[s]).wait()

    def wait_store(s):
        pltpu.make_async_copy(outs[s], of.at[pl.ds(0, _C)], sos[s]).wait()

    def compute(s):
        @plsc.parallel_loop(0, _C)
        def _(r):
            for k in range(D_MODEL // _L):
                sl = pl.ds(k * _L, _L)
                plsc.addupdate(outs[s].at[r, sl], pes[s][r, sl])

    issue_loads(0, 0)

    def pair_body(j2, carry):
        c0 = j2 * 2

        wait_loads(0)
        compute(0)
        pltpu.async_copy(outs[0], of.at[pl.ds(rbase_w + c0 * _C, _C)], sos[0])

        @pl.when(j2 >= 1)
        def _():
            wait_store(1)
        issue_loads(c0 + 1, 1)

        wait_loads(1)
        compute(1)
        pltpu.async_copy(
            outs[1], of.at[pl.ds(rbase_w + (c0 + 1) * _C, _C)], sos[1])

        @pl.when(j2 < _NPAIR - 1)
        def _():
            wait_store(0)
            issue_loads(c0 + 2, 0)
        return carry

    lax.fori_loop(0, _NPAIR, pair_body, 0)

    wait_store(0)
    wait_store(1)


@jax.jit
def _pe_add(pe, idx3, x):
    f = functools.partial(
        pl.kernel,
        mesh=plsc.VectorSubcoreMesh(core_axis_name="c", subcore_axis_name="s"),
        out_type=jax.ShapeDtypeStruct((SEQ, BATCH, D_MODEL), jnp.float32),
        scratch_types=[
            pltpu.VMEM((_CHUNKS, _C), jnp.int32),
            pltpu.VMEM((_C, D_MODEL), jnp.float32),
            pltpu.VMEM((_C, D_MODEL), jnp.float32),
            pltpu.VMEM((_C, D_MODEL), jnp.float32),
            pltpu.VMEM((_C, D_MODEL), jnp.float32),
            pltpu.SemaphoreType.DMA,
            pltpu.SemaphoreType.DMA,
            pltpu.SemaphoreType.DMA,
            pltpu.SemaphoreType.DMA,
            pltpu.SemaphoreType.DMA,
            pltpu.SemaphoreType.DMA,
        ],
    )(_sc_body)
    return f(pe, idx3, x)


def kernel(x, step, pe):
    idx3 = step.reshape(_NW, _CHUNKS, _C).astype(jnp.int32)
    return _pe_add(pe, idx3, x)


# speedup vs baseline: 1.3861x; 1.3861x over previous
import functools

import jax
import jax.numpy as jnp
from jax import lax
from jax.experimental import pallas as pl
from jax.experimental.pallas import tpu as pltpu
from jax.experimental.pallas import tpu_sc as plsc

SEQ = 4096
BATCH = 4
D_MODEL = 1024
N_ROWS = SEQ * BATCH
_L = 16
_NC, _NS = 2, 16
_NW = _NC * _NS
_B_PER_W = N_ROWS // _NW
_C = 8
_CHUNKS = _B_PER_W // _C
_NSLOT = 4
_NQ = _CHUNKS // _NSLOT


def _sc_body(pe_hbm, idx_hbm, x_hbm, out_hbm, idx_v,
             x_v0, x_v1, x_v2, x_v3,
             pe_v0, pe_v1, pe_v2, pe_v3,
             o_v0, o_v1, o_v2, o_v3,
             sx0, sx1, sx2, sx3,
             sg0, sg1, sg2, sg3,
             so0, so1, so2, so3):
    wid = lax.axis_index("s") * _NC + lax.axis_index("c")
    xf = x_hbm.reshape(N_ROWS, D_MODEL)
    of = out_hbm.reshape(N_ROWS, D_MODEL)
    rbase_w = wid * _B_PER_W
    pltpu.sync_copy(idx_hbm.at[wid], idx_v)

    xs = (x_v0, x_v1, x_v2, x_v3)
    pes = (pe_v0, pe_v1, pe_v2, pe_v3)
    outs = (o_v0, o_v1, o_v2, o_v3)
    sxs = (sx0, sx1, sx2, sx3)
    sgs = (sg0, sg1, sg2, sg3)
    sos = (so0, so1, so2, so3)

    def issue_loads(c, s):
        pltpu.async_copy(xf.at[pl.ds(rbase_w + c * _C, _C)], xs[s], sxs[s])
        pltpu.async_copy(pe_hbm.at[idx_v.at[c]], pes[s], sgs[s])

    def wait_loads(s):
        pltpu.make_async_copy(xf.at[pl.ds(0, _C)], xs[s], sxs[s]).wait()
        pltpu.make_async_copy(pe_hbm.at[idx_v.at[0]], pes[s], sgs[s]).wait()

    def wait_store(s):
        pltpu.make_async_copy(outs[s], of.at[pl.ds(0, _C)], sos[s]).wait()

    def compute(s):
        @plsc.parallel_loop(0, _C)
        def _(r):
            for k in range(D_MODEL // _L):
                sl = pl.ds(k * _L, _L)
                outs[s][r, sl] = xs[s][r, sl] + pes[s][r, sl]

    for s in range(_NSLOT - 1):
        issue_loads(s, s)

    def quad_body(j4, carry):
        for s in range(_NSLOT):
            c = j4 * _NSLOT + s
            wait_loads(s)
            compute(s)
            pltpu.async_copy(
                outs[s], of.at[pl.ds(rbase_w + c * _C, _C)], sos[s])

            sp = (s + _NSLOT - 1) % _NSLOT
            if s == 0:
                @pl.when(j4 >= 1)
                def _():
                    wait_store(sp)
                issue_loads(c + 3, sp)
            else:
                @pl.when(j4 < _NQ - 1)
                def _():
                    wait_store(sp)
                    issue_loads(c + 3, sp)
        return carry

    lax.fori_loop(0, _NQ, quad_body, 0)

    for s in range(_NSLOT):
        wait_store(s)


@jax.jit
def _pe_add(pe, idx3, x):
    f = functools.partial(
        pl.kernel,
        mesh=plsc.VectorSubcoreMesh(core_axis_name="c", subcore_axis_name="s"),
        out_type=jax.ShapeDtypeStruct((SEQ, BATCH, D_MODEL), jnp.float32),
        scratch_types=(
            [pltpu.VMEM((_CHUNKS, _C), jnp.int32)]
            + [pltpu.VMEM((_C, D_MODEL), jnp.float32)] * 12
            + [pltpu.SemaphoreType.DMA] * 12
        ),
    )(_sc_body)
    return f(pe, idx3, x)


def kernel(x, step, pe):
    idx3 = step.reshape(_NW, _CHUNKS, _C).astype(jnp.int32)
    return _pe_add(pe, idx3, x)


# speedup vs baseline: 1.4272x; 1.0297x over previous
import functools

import jax
import jax.numpy as jnp
from jax import lax
from jax.experimental import pallas as pl
from jax.experimental.pallas import tpu as pltpu
from jax.experimental.pallas import tpu_sc as plsc

SEQ = 4096
BATCH = 4
D_MODEL = 1024
_L = 16
_NC, _NS = 2, 16
_NW = _NC * _NS
_S_PER_W = SEQ // _NW
_CS = 4
_C = _CS * BATCH
_CW = _C * D_MODEL
_CHUNKS = _S_PER_W // _CS
_NPAIR = _CHUNKS // 2


def _sc_body(pe_hbm, idx_hbm, x_hbm, out_hbm, idx_v,
             x_v0, x_v1, pe_v0, pe_v1, o_v0, o_v1,
             sx0, sx1, sg0, sg1, so0, so1):
    wid = lax.axis_index("s") * _NC + lax.axis_index("c")
    xf = x_hbm.reshape(SEQ * BATCH, D_MODEL)
    of = out_hbm.reshape(SEQ * BATCH, D_MODEL)
    rbase_w = wid * _S_PER_W * BATCH
    pltpu.sync_copy(idx_hbm.at[wid], idx_v)

    xs = (x_v0, x_v1)
    pes = (pe_v0, pe_v1)
    outs = (o_v0, o_v1)
    sxs = (sx0, sx1)
    sgs = (sg0, sg1)
    sos = (so0, so1)

    def issue_loads(c, s):
        pltpu.async_copy(
            xf.at[pl.ds(rbase_w + c * _C, _C)], xs[s], sxs[s])
        pltpu.async_copy(pe_hbm.at[idx_v.at[c]], pes[s], sgs[s])

    issue_loads(0, 0)
    issue_loads(1, 1)

    def pair_body(j2, carry):
        for s in (0, 1):
            c = j2 * 2 + s
            pltpu.make_async_copy(
                xf.at[pl.ds(0, _C)], xs[s], sxs[s]).wait()
            pltpu.make_async_copy(
                pe_hbm.at[idx_v.at[0]], pes[s], sgs[s]).wait()

            @pl.when(j2 >= 1)
            def _():
                pltpu.make_async_copy(
                    outs[s], of.at[pl.ds(0, _C)], sos[s]).wait()

            @plsc.parallel_loop(0, _C)
            def _(r):
                for k in range(D_MODEL // _L):
                    sl = pl.ds(k * _L, _L)
                    outs[s][r, sl] = xs[s][r, sl] + pes[s][r, sl]

            pltpu.async_copy(
                outs[s], of.at[pl.ds(rbase_w + c * _C, _C)], sos[s])

            @pl.when(j2 < _NPAIR - 1)
            def _():
                issue_loads(c + 2, s)
        return carry

    lax.fori_loop(0, _NPAIR, pair_body, 0)

    for s in (0, 1):
        pltpu.make_async_copy(
            outs[s], of.at[pl.ds(0, _C)], sos[s]).wait()


@jax.jit
def _pe_add(pe, idx3, x):
    f = functools.partial(
        pl.kernel,
        mesh=plsc.VectorSubcoreMesh(core_axis_name="c", subcore_axis_name="s"),
        out_type=jax.ShapeDtypeStruct((SEQ, BATCH, D_MODEL), jnp.float32),
        scratch_types=[
            pltpu.VMEM((_CHUNKS, _C), jnp.int32),
            pltpu.VMEM((_C, D_MODEL), jnp.float32),
            pltpu.VMEM((_C, D_MODEL), jnp.float32),
            pltpu.VMEM((_C, D_MODEL), jnp.float32),
            pltpu.VMEM((_C, D_MODEL), jnp.float32),
            pltpu.VMEM((_C, D_MODEL), jnp.float32),
            pltpu.VMEM((_C, D_MODEL), jnp.float32),
            pltpu.SemaphoreType.DMA,
            pltpu.SemaphoreType.DMA,
            pltpu.SemaphoreType.DMA,
            pltpu.SemaphoreType.DMA,
            pltpu.SemaphoreType.DMA,
            pltpu.SemaphoreType.DMA,
        ],
    )(_sc_body)
    return f(pe, idx3, x)


def kernel(x, step, pe):
    idx3 = step.reshape(_NW, _CHUNKS, _C).astype(jnp.int32)
    return _pe_add(pe, idx3, x)
